# baseline (speedup 1.0000x reference)
import functools

import jax
import jax.numpy as jnp
from jax import lax
from jax.experimental import pallas as pl
from jax.experimental.pallas import tpu as pltpu
from jax.experimental.pallas import tpu_sc as plsc

N = 10000
E = 320000
D = 128
H = 16

NC = 2
NS = 16
NW = NC * NS
BLK = 128
EP = ((E + NW * BLK - 1) // (NW * BLK)) * (NW * BLK)
R = EP // BLK
RPT = R // NW
N2 = 10240
NPS2 = N2 // NS
BN = N2 // 8

_mesh = functools.partial(
    plsc.VectorSubcoreMesh, core_axis_name="c", subcore_axis_name="s")

_sc_params = functools.partial(
    pltpu.CompilerParams, needs_layout_passes=False, use_tc_tiling_on_sc=False)


def _wid():
    return lax.axis_index("c") * NS + lax.axis_index("s")


def _mm(a, b):
    return lax.dot_general(a, b, (((1,), (0,)), ((), ())),
                           preferred_element_type=jnp.float32)


def _deg_call(src2d, w2d, z1):
    @functools.partial(
        pl.kernel,
        out_type=jax.ShapeDtypeStruct((NC, N2), jnp.float32),
        mesh=_mesh(),
        compiler_params=_sc_params(),
        scratch_types=[
            pltpu.VMEM_SHARED((N2,), jnp.float32),
            pltpu.VMEM((RPT, BLK), jnp.int32),
            pltpu.VMEM((RPT, BLK), jnp.float32),
            pltpu.VMEM((BLK,), jnp.float32),
            pltpu.SemaphoreType.DMA,
        ],
    )
    def k(src_h, w_h, z_h, out_h, shared, si_a, w_a, drow, sem):
        c = lax.axis_index("c")
        s = lax.axis_index("s")
        base = _wid() * RPT
        pltpu.sync_copy(z_h.at[pl.ds(s * NPS2, NPS2)],
                        shared.at[pl.ds(s * NPS2, NPS2)])
        pltpu.sync_copy(src_h.at[pl.ds(base, RPT)], si_a)
        pltpu.sync_copy(w_h.at[pl.ds(base, RPT)], w_a)
        plsc.subcore_barrier()

        def issue(r, _):
            pltpu.async_copy(w_a.at[r], shared.at[si_a.at[r]], sem, add=True)
            return _

        lax.fori_loop(0, RPT, issue, None)

        def drain(r, _):
            pltpu.make_async_copy(z_h.at[pl.ds(0, BLK)], drow, sem).wait()
            return _

        lax.fori_loop(0, RPT, drain, None)
        plsc.subcore_barrier()
        pltpu.sync_copy(shared.at[pl.ds(s * NPS2, NPS2)],
                        out_h.at[c, pl.ds(s * NPS2, NPS2)])

    return k(src2d, w2d, z1)


def _tc_dis(deg_p):
    def body(deg_ref, dis_ref):
        d = deg_ref[0, :] + deg_ref[1, :]
        dis_ref[...] = jnp.where(d > 0.0, 1.0 / jnp.sqrt(d), 0.0)

    return pl.pallas_call(
        body,
        out_shape=jax.ShapeDtypeStruct((N2,), jnp.float32),
    )(deg_p)


def _norm_call(dis1d, src2d, dst2d, w2d):
    @functools.partial(
        pl.kernel,
        out_type=jax.ShapeDtypeStruct((R, BLK), jnp.float32),
        mesh=_mesh(),
        compiler_params=_sc_params(),
        scratch_types=[
            pltpu.VMEM((N2,), jnp.float32),
            pltpu.VMEM((RPT, BLK), jnp.int32),
            pltpu.VMEM((RPT, BLK), jnp.int32),
            pltpu.VMEM((RPT, BLK), jnp.float32),
            pltpu.VMEM((RPT, BLK), jnp.float32),
        ],
    )
    def k(dis_h, src_h, dst_h, w_h, out_h, dis_v, si_a, di_a, w_a, o_a):
        base = _wid() * RPT
        pltpu.sync_copy(dis_h, dis_v)
        pltpu.sync_copy(src_h.at[pl.ds(base, RPT)], si_a)
        pltpu.sync_copy(dst_h.at[pl.ds(base, RPT)], di_a)
        pltpu.sync_copy(w_h.at[pl.ds(base, RPT)], w_a)

        def body(r, _):
            for j in range(BLK // 16):
                sl = pl.ds(j * 16, 16)
                ds_ = plsc.load_gather(dis_v, [si_a[r, sl]])
                dd_ = plsc.load_gather(dis_v, [di_a[r, sl]])
                o_a[r, sl] = -(ds_ * w_a[r, sl] * dd_)
            return _

        lax.fori_loop(0, RPT, body, None)
        pltpu.sync_copy(o_a, out_h.at[pl.ds(base, RPT)])

    return k(dis1d, src2d, dst2d, w2d)


def _prop_call(table, src2d, dst2d, nn2d, zF, F, blk=BLK):
    fuse_norm = False
    rpt = (EP // blk) // NW
    out_type = jax.ShapeDtypeStruct((NC, N2, F), jnp.float32)

    @functools.partial(
        pl.kernel,
        out_type=out_type,
        mesh=_mesh(),
        compiler_params=_sc_params(),
        scratch_types=[
            pltpu.VMEM_SHARED((N2, F), jnp.float32),
            pltpu.VMEM((rpt, blk), jnp.int32),
            pltpu.VMEM((rpt, blk), jnp.int32),
            pltpu.VMEM((rpt, blk), jnp.float32),
            pltpu.VMEM((2, blk, F), jnp.float32),
            pltpu.SemaphoreType.DMA,
            pltpu.SemaphoreType.DMA,
            pltpu.SemaphoreType.DMA,
            pltpu.SemaphoreType.DMA,
        ],
    )
    def k(tab_h, src_h, dst_h, *rest):
        (nn_h, z_h, out_h,
         shared, si_a, di_a, nn_a, gbuf,
         sg0, sg1, ss0, ss1) = rest
        c = lax.axis_index("c")
        s = lax.axis_index("s")
        base = _wid() * rpt
        pltpu.sync_copy(z_h.at[pl.ds(s * NPS2, NPS2)],
                        shared.at[pl.ds(s * NPS2, NPS2)])
        pltpu.sync_copy(src_h.at[pl.ds(base, rpt)], si_a)
        pltpu.sync_copy(dst_h.at[pl.ds(base, rpt)], di_a)
        pltpu.sync_copy(nn_h.at[pl.ds(base, rpt)], nn_a)
        plsc.subcore_barrier()

        sems_g = (sg0, sg1)
        sems_s = (ss0, ss1)

        def scale_group(r, cur, g):
            b16 = g * 16
            nnvec = nn_a[r, pl.ds(b16, 16)]
            for i in range(16):
                sc = nnvec[i]
                for j in range(F // 16):
                    sl = pl.ds(j * 16, 16)
                    gbuf[cur, b16 + i, sl] = gbuf[cur, b16 + i, sl] * sc

        def stage(r, cur, nxt):
            @pl.when(r >= 1)
            def _():
                pltpu.make_async_copy(z_h.at[pl.ds(0, blk)], gbuf.at[nxt],
                                      sems_s[nxt]).wait()

            @pl.when(r + 1 < rpt)
            def _():
                pltpu.async_copy(tab_h.at[si_a.at[r + 1]], gbuf.at[nxt],
                                 sems_g[nxt])

            pltpu.make_async_copy(tab_h.at[pl.ds(0, blk)], gbuf.at[cur],
                                  sems_g[cur]).wait()

            if F >= 64:
                def sc_body(g, _):
                    scale_group(r, cur, g)
                    return _

                lax.fori_loop(0, blk // 16, sc_body, None)
            else:
                for g in range(blk // 16):
                    scale_group(r, cur, g)
            pltpu.async_copy(gbuf.at[cur], shared.at[di_a.at[r]],
                             sems_s[cur], add=True)

        pltpu.async_copy(tab_h.at[si_a.at[0]], gbuf.at[0], sg0)

        def body(kk, _):
            r = kk * 2
            stage(r, 0, 1)

            @pl.when(r + 1 < rpt)
            def _():
                stage(r + 1, 1, 0)

            return _

        lax.fori_loop(0, (rpt + 1) // 2, body, None)
        pltpu.make_async_copy(z_h.at[pl.ds(0, blk)], gbuf.at[(rpt - 1) % 2],
                              sems_s[(rpt - 1) % 2]).wait()
        plsc.subcore_barrier()
        pltpu.sync_copy(shared.at[pl.ds(s * NPS2, NPS2)],
                        out_h.at[c, pl.ds(s * NPS2, NPS2)])

    return k(table, src2d, dst2d, nn2d, zF)


def _prop2_call(p1a, p1b, src2d, dst2d, nn2d, z16):
    @functools.partial(
        pl.kernel,
        out_type=jax.ShapeDtypeStruct((NC, N2, 16), jnp.float32),
        mesh=_mesh(),
        compiler_params=_sc_params(),
        scratch_types=[
            pltpu.VMEM_SHARED((N2, 16), jnp.float32),
            pltpu.VMEM((RPT, BLK), jnp.int32),
            pltpu.VMEM((RPT, BLK), jnp.int32),
            pltpu.VMEM((RPT, BLK), jnp.float32),
            pltpu.VMEM((2, BLK, 16), jnp.float32),
            pltpu.VMEM((2, BLK, 16), jnp.float32),
            pltpu.VMEM((2, BLK, 16), jnp.float32),
            pltpu.SemaphoreType.DMA,
            pltpu.SemaphoreType.DMA,
            pltpu.SemaphoreType.DMA,
            pltpu.SemaphoreType.DMA,
        ],
    )
    def k(pa_h, pb_h, src_h, dst_h, nn_h, z_h, out_h,
          shared, si_a, di_a, nn_a, ga, gb, sbuf, sg0, sg1, ss0, ss1):
        c = lax.axis_index("c")
        s = lax.axis_index("s")
        base = _wid() * RPT
        pltpu.sync_copy(z_h.at[pl.ds(s * NPS2, NPS2)],
                        shared.at[pl.ds(s * NPS2, NPS2)])
        pltpu.sync_copy(src_h.at[pl.ds(base, RPT)], si_a)
        pltpu.sync_copy(dst_h.at[pl.ds(base, RPT)], di_a)
        pltpu.sync_copy(nn_h.at[pl.ds(base, RPT)], nn_a)
        plsc.subcore_barrier()

        sems_g = (sg0, sg1)
        sems_s = (ss0, ss1)

        def issue_gathers(r, buf):
            pltpu.async_copy(pa_h.at[si_a.at[r]], ga.at[buf], sems_g[buf])
            pltpu.async_copy(pb_h.at[si_a.at[r]], gb.at[buf], sems_g[buf])

        def stage(r, cur, nxt):
            @pl.when(r + 1 < RPT)
            def _():
                issue_gathers(r + 1, nxt)

            pltpu.make_async_copy(pa_h.at[pl.ds(0, BLK)], ga.at[cur],
                                  sems_g[cur]).wait()
            pltpu.make_async_copy(pa_h.at[pl.ds(0, BLK)], gb.at[cur],
                                  sems_g[cur]).wait()

            @pl.when(r >= 2)
            def _():
                pltpu.make_async_copy(z_h.at[pl.ds(0, BLK)], sbuf.at[cur],
                                      sems_s[cur]).wait()

            for g in range(BLK // 16):
                b16 = g * 16
                nnvec = nn_a[r, pl.ds(b16, 16)]
                for i in range(16):
                    sc = nnvec[i]
                    sbuf[cur, b16 + i, :] = (ga[cur, b16 + i, :]
                                             + gb[cur, b16 + i, :]) * sc
            pltpu.async_copy(sbuf.at[cur], shared.at[di_a.at[r]],
                             sems_s[cur], add=True)

        issue_gathers(0, 0)

        def body(kk, _):
            r = kk * 2
            stage(r, 0, 1)

            @pl.when(r + 1 < RPT)
            def _():
                stage(r + 1, 1, 0)

            return _

        lax.fori_loop(0, (RPT + 1) // 2, body, None)
        pltpu.make_async_copy(z_h.at[pl.ds(0, BLK)], sbuf.at[(RPT - 2) % 2],
                              sems_s[(RPT - 2) % 2]).wait()
        pltpu.make_async_copy(z_h.at[pl.ds(0, BLK)], sbuf.at[(RPT - 1) % 2],
                              sems_s[(RPT - 1) % 2]).wait()
        plsc.subcore_barrier()
        pltpu.sync_copy(shared.at[pl.ds(s * NPS2, NPS2)],
                        out_h.at[c, pl.ds(s * NPS2, NPS2)])

    return k(p1a, p1b, src2d, dst2d, nn2d, z16)


def _tc_sum(p1):
    def body(p_ref, o_ref):
        o_ref[...] = p_ref[0] + p_ref[1]

    return pl.pallas_call(
        body,
        grid=(8,),
        in_specs=[pl.BlockSpec((NC, BN, D), lambda i: (0, i, 0))],
        out_specs=pl.BlockSpec((BN, D), lambda i: (i, 0)),
        out_shape=jax.ShapeDtypeStruct((N2, D), jnp.float32),
    )(p1)


def _tc_cheb(act, p1a, p1b, p2a, p2b, W0, W1, W2, brow, Fin, relu):
    def body(a_ref, p1a_ref, p1b_ref, p2a_ref, p2b_ref,
             w0_ref, w1_ref, w2_ref, b_ref, o_ref):
        a = a_ref[...]
        tx1 = p1a_ref[...] + p1b_ref[...]
        tx2 = 2.0 * (p2a_ref[...] + p2b_ref[...]) - a
        out = _mm(a, w0_ref[...])
        out = out + _mm(tx1, w1_ref[...])
        out = out + _mm(tx2, w2_ref[...])
        out = out + b_ref[...]
        if relu:
            out = jnp.maximum(out, 0.0)
        o_ref[...] = out

    return pl.pallas_call(
        body,
        grid=(8,),
        in_specs=[pl.BlockSpec((BN, Fin), lambda i: (i, 0)),
                  pl.BlockSpec((BN, Fin), lambda i: (i, 0)),
                  pl.BlockSpec((BN, Fin), lambda i: (i, 0)),
                  pl.BlockSpec((BN, Fin), lambda i: (i, 0)),
                  pl.BlockSpec((BN, Fin), lambda i: (i, 0)),
                  pl.BlockSpec((Fin, H), lambda i: (0, 0)),
                  pl.BlockSpec((Fin, H), lambda i: (0, 0)),
                  pl.BlockSpec((Fin, H), lambda i: (0, 0)),
                  pl.BlockSpec((1, H), lambda i: (0, 0))],
        out_specs=pl.BlockSpec((BN, H), lambda i: (i, 0)),
        out_shape=jax.ShapeDtypeStruct((N2, H), jnp.float32),
    )(act, p1a, p1b, p2a, p2b, W0, W1, W2, brow)


def _tc_final(h2, p1a, p1b, p2a, p2b, W3, b3):
    def body(a_ref, p1a_ref, p1b_ref, p2a_ref, p2b_ref, w_ref, b_ref,
             choice_ref, v_ref):
        a = a_ref[...]
        tx1 = p1a_ref[...] + p1b_ref[...]
        tx2 = 2.0 * (p2a_ref[...] + p2b_ref[...]) - a
        cfull = _mm(a, w_ref[0]) + _mm(tx1, w_ref[1]) + _mm(tx2, w_ref[2])
        cfull = cfull + b_ref[0, 0]
        valid = lax.broadcasted_iota(jnp.int32, (N2, 1), 0) < N
        cm = jnp.where(valid, cfull, -jnp.inf)
        m = jnp.max(cm)
        ex = jnp.exp(cm - m)
        choice_ref[...] = ex / jnp.sum(ex)
        v_ref[...] = jnp.sum(jnp.where(valid, a, 0.0), axis=0,
                             keepdims=True) / N

    return pl.pallas_call(
        body,
        out_shape=(jax.ShapeDtypeStruct((N2, 1), jnp.float32),
                   jax.ShapeDtypeStruct((1, H), jnp.float32)),
    )(h2, p1a, p1b, p2a, p2b, W3, b3)


def kernel(x, edge_index, weight, W1, b1, W2, b2, W3, b3, A2w, A2b):
    pad = EP - E
    src = jnp.pad(edge_index[0], (0, pad)).reshape(R, BLK)
    dst = jnp.pad(edge_index[1], (0, pad)).reshape(R, BLK)
    w2d = jnp.pad(weight, (0, pad)).reshape(R, BLK)

    z1 = jnp.zeros((N2,), jnp.float32)
    z16 = jnp.zeros((N2, 16), jnp.float32)
    z128 = jnp.zeros((N2, D), jnp.float32)

    deg_p = _deg_call(src, w2d, z1)
    dis1d = _tc_dis(deg_p)

    xp = jnp.pad(x, ((0, N2 - N), (0, 0)))

    nn2d = _norm_call(dis1d, src, dst, w2d)

    src64 = src.reshape(-1, 64)
    dst64 = dst.reshape(-1, 64)
    nn64 = nn2d.reshape(-1, 64)
    p1 = _prop_call(xp, src64, dst64, nn64, z128, D, blk=64)
    tx1 = _tc_sum(p1)
    p2 = _prop_call(tx1, src64, dst64, nn64, z128, D, blk=64)
    h1 = _tc_cheb(xp, p1[0], p1[1], p2[0], p2[1],
                  W1[0], W1[1], W1[2], b1.reshape(1, H), D, True)

    p1 = _prop_call(h1, src, dst, nn2d, z16, 16)
    p2 = _prop2_call(p1[0], p1[1], src, dst, nn2d, z16)
    h2 = _tc_cheb(h1, p1[0], p1[1], p2[0], p2[1],
                  W2[0], W2[1], W2[2], b2.reshape(1, H), H, True)

    p1 = _prop_call(h2, src, dst, nn2d, z16, 16)
    p2 = _prop2_call(p1[0], p1[1], src, dst, nn2d, z16)
    choice, v = _tc_final(h2, p1[0], p1[1], p2[0], p2[1], W3,
                          b3.reshape(1, 1))
    value = (jnp.dot(v, A2w.T) + A2b).squeeze()
    return choice[:N, 0], value
